# baseline (speedup 1.0000x reference)
import functools

import jax
import jax.numpy as jnp
from jax import lax
from jax.experimental import pallas as pl
from jax.experimental.pallas import tpu as pltpu
from jax.experimental.pallas import tpu_sc as plsc

_N = 10000
_NPAD = 10240
_E = 320000
_NG = 16
_EPS = 1e-5

_C = 125
_NCH_TOT = _E // _C
_NW = 32
_NCHW = _NCH_TOT // _NW
_RPS = _NPAD // 16
_ZC = 128
_DEG_D = 16

_R = 2000
_GRID = _N // _R


def _fill_f32(buf, rows, d, val):
    vals = jnp.full((16,), val, jnp.float32)

    def row_body(i, _):
        def col_body(k, _):
            buf[i, pl.ds(k * 16, 16)] = vals
            return 0
        return lax.fori_loop(0, d // 16, col_body, 0)

    lax.fori_loop(0, rows, row_body, 0)


def _sc_edge_body(D, nbuf, deg_mode, fused, u_hbm, col_hbm, row_hbm,
                  out_hbm, refs):
    nch = 2 * _NCHW if fused else _NCHW
    rowi, coli = refs[0], refs[1]
    bufs = list(refs[2:2 + nbuf])
    acc = refs[2 + nbuf]
    gsems = list(refs[3 + nbuf:3 + 2 * nbuf])
    ssems = list(refs[3 + 2 * nbuf:3 + 3 * nbuf])

    c = lax.axis_index("c")
    s = lax.axis_index("s")
    idx_base = s * nch if fused else (s * 2 + c) * nch
    table = u_hbm.at[c] if fused else u_hbm

    ld0 = pltpu.async_copy(col_hbm.at[pl.ds(idx_base, nch)], coli, ssems[0])
    if not deg_mode:
        ld1 = pltpu.async_copy(row_hbm.at[pl.ds(idx_base, nch)], rowi,
                               ssems[1])

    _fill_f32(bufs[0], _ZC, D, 0.0)
    for k in range(_RPS // _ZC):
        pltpu.sync_copy(bufs[0], acc.at[pl.ds(s * _RPS + k * _ZC, _ZC)])
    ld0.wait()
    if not deg_mode:
        ld1.wait()
    plsc.subcore_barrier()

    b0s = [buf.at[pl.ds(0, _C)] for buf in bufs]

    if deg_mode:
        _fill_f32(bufs[0], _C, D, 1.0)

        def grp(it, _):
            base = it * nbuf
            for b in range(nbuf):
                pltpu.async_copy(b0s[0], acc.at[coli.at[base + b]],
                                 ssems[b], add=True)
            for b in range(nbuf):
                pltpu.make_async_copy(
                    b0s[0], acc.at[coli.at[0]], ssems[b]).wait()
            return 0
    else:
        for b in range(nbuf):
            pltpu.async_copy(table.at[rowi.at[b]], b0s[b], gsems[b])

        def grp(it, _):
            base = it * nbuf
            for b in range(nbuf):
                pltpu.make_async_copy(
                    table.at[rowi.at[0]], b0s[b], gsems[b]).wait()
                pltpu.async_copy(b0s[b], acc.at[coli.at[base + b]],
                                 ssems[b], add=True)
            nxt = base + nbuf
            for b in range(nbuf):
                pltpu.make_async_copy(
                    b0s[b], acc.at[coli.at[0]], ssems[b]).wait()

                @pl.when(nxt + b < nch)
                def _():
                    pltpu.async_copy(table.at[rowi.at[nxt + b]],
                                     b0s[b], gsems[b])
            return 0

    lax.fori_loop(0, nch // nbuf, grp, 0)
    plsc.subcore_barrier()

    sl = pl.ds(s * _RPS, _RPS)
    pltpu.sync_copy(acc.at[sl], out_hbm.at[c].at[sl])


def _sc_scratch(D, nbuf, nch):
    return ([
        pltpu.VMEM((nch, _C), jnp.int32),
        pltpu.VMEM((nch, _C), jnp.int32),
    ] + [pltpu.VMEM((_ZC, D), jnp.float32)] * nbuf
      + [pltpu.VMEM_SHARED((_NPAD, D), jnp.float32)]
      + [pltpu.SemaphoreType.DMA] * (2 * nbuf))


_FUSED_NBUF = {64: 5, 32: 8, 16: 8}
_Q_NBUF = 8


@functools.cache
def _make_sc_pass(D, fused):
    mesh = plsc.VectorSubcoreMesh(core_axis_name="c", subcore_axis_name="s")
    nbuf = _FUSED_NBUF[D] if fused else _Q_NBUF
    nch = 2 * _NCHW if fused else _NCHW

    @functools.partial(
        pl.kernel, mesh=mesh,
        out_type=jax.ShapeDtypeStruct((2, _NPAD, D), jnp.float32),
        scratch_types=_sc_scratch(D, nbuf, nch),
        compiler_params=pltpu.CompilerParams(use_tc_tiling_on_sc=False),
        name=f"sc_edge_pass_{D}_{'ab' if fused else 'q'}",
    )
    def sc_pass(u_hbm, row_hbm, col_hbm, out_hbm, *refs):
        _sc_edge_body(D, nbuf, False, fused, u_hbm, col_hbm, row_hbm,
                      out_hbm, refs)

    return sc_pass


@functools.cache
def _make_sc_deg():
    mesh = plsc.VectorSubcoreMesh(core_axis_name="c", subcore_axis_name="s")
    D = _DEG_D

    @functools.partial(
        pl.kernel, mesh=mesh,
        out_type=jax.ShapeDtypeStruct((2, _NPAD, D), jnp.float32),
        scratch_types=_sc_scratch(D, _Q_NBUF, _NCHW),
        compiler_params=pltpu.CompilerParams(use_tc_tiling_on_sc=False),
        name="sc_deg_pass",
    )
    def sc_deg(row_hbm, out_hbm, *refs):
        _sc_edge_body(D, _Q_NBUF, True, False, None, row_hbm, None,
                      out_hbm, refs)

    return sc_deg


def _lrelu(z):
    return jnp.where(z >= 0, z, 0.01 * z)


def _tc_start(x, degp, d_half):
    def body(x_ref, d_ref, u_ref, dis_ref):
        deg = d_ref[0, :, 0:1] + d_ref[1, :, 0:1]
        dis = jnp.where(deg > 0, lax.rsqrt(deg), 0.0)
        u_ref[0] = dis * x_ref[:, :d_half]
        u_ref[1] = dis * x_ref[:, d_half:]
        dis_ref[...] = dis

    f_in = x.shape[1]
    return pl.pallas_call(
        body,
        grid=(_GRID,),
        in_specs=[
            pl.BlockSpec((_R, f_in), lambda i: (i, 0)),
            pl.BlockSpec((2, _R, _DEG_D), lambda i: (0, i, 0)),
        ],
        out_specs=[
            pl.BlockSpec((2, _R, d_half), lambda i: (0, i, 0)),
            pl.BlockSpec((_R, 1), lambda i: (i, 0)),
        ],
        out_shape=[
            jax.ShapeDtypeStruct((2, _N, d_half), jnp.float32),
            jax.ShapeDtypeStruct((_N, 1), jnp.float32),
        ],
    )(x, degp)


def _tc_mid_halves(p, dis, d_half):
    def body(p_ref, dis_ref, t1_ref, u2_ref):
        dis_v = dis_ref[...]
        t1a = -dis_v * p_ref[0]
        t1b = -dis_v * p_ref[1]
        t1_ref[:, :d_half] = t1a
        t1_ref[:, d_half:] = t1b
        u2_ref[0] = dis_v * t1a
        u2_ref[1] = dis_v * t1b

    return pl.pallas_call(
        body,
        grid=(_GRID,),
        in_specs=[
            pl.BlockSpec((2, _R, d_half), lambda i: (0, i, 0)),
            pl.BlockSpec((_R, 1), lambda i: (i, 0)),
        ],
        out_specs=[
            pl.BlockSpec((_R, 2 * d_half), lambda i: (i, 0)),
            pl.BlockSpec((2, _R, d_half), lambda i: (0, i, 0)),
        ],
        out_shape=[
            jax.ShapeDtypeStruct((_N, 2 * d_half), jnp.float32),
            jax.ShapeDtypeStruct((2, _N, d_half), jnp.float32),
        ],
    )(p, dis)


def _tc_mid(p, dis, d):
    def body(p_ref, dis_ref, t1_ref, u2_ref):
        dis_v = dis_ref[...]
        t1 = -dis_v * (p_ref[0] + p_ref[1])
        t1_ref[...] = t1
        u2_ref[...] = dis_v * t1

    return pl.pallas_call(
        body,
        grid=(_GRID,),
        in_specs=[
            pl.BlockSpec((2, _R, d), lambda i: (0, i, 0)),
            pl.BlockSpec((_R, 1), lambda i: (i, 0)),
        ],
        out_specs=[
            pl.BlockSpec((_R, d), lambda i: (i, 0)),
            pl.BlockSpec((_R, d), lambda i: (i, 0)),
        ],
        out_shape=[
            jax.ShapeDtypeStruct((_N, d), jnp.float32),
            jax.ShapeDtypeStruct((_N, d), jnp.float32),
        ],
    )(p, dis)


def _cheb_block(h, t1, p2_ref, dis_v, w_ref, sc_ref, sh_ref, halves, d):
    if halves:
        v2 = jnp.concatenate([p2_ref[0], p2_ref[1]], axis=1)
    else:
        v2 = p2_ref[0] + p2_ref[1]
    tx2 = 2.0 * (-dis_v * v2) - h
    mm = (jnp.dot(h, w_ref[0], preferred_element_type=jnp.float32)
          + jnp.dot(t1, w_ref[1], preferred_element_type=jnp.float32)
          + jnp.dot(tx2, w_ref[2], preferred_element_type=jnp.float32))
    return _lrelu(mm * sc_ref[...] + sh_ref[...])


def _tc_end(h, t1, p2, dis, wstk, scale, shift, halves, d, d_out):
    d_p = d // 2 if halves else d

    def body(h_ref, t1_ref, p2_ref, dis_ref, w_ref, sc_ref, sh_ref,
             hn_ref, un_ref):
        dis_v = dis_ref[...]
        hn = _cheb_block(h_ref[...], t1_ref[...], p2_ref, dis_v, w_ref,
                         sc_ref, sh_ref, halves, d)
        hn_ref[...] = hn
        un_ref[...] = dis_v * hn

    return pl.pallas_call(
        body,
        grid=(_GRID,),
        in_specs=[
            pl.BlockSpec((_R, d), lambda i: (i, 0)),
            pl.BlockSpec((_R, d), lambda i: (i, 0)),
            pl.BlockSpec((2, _R, d_p), lambda i: (0, i, 0)),
            pl.BlockSpec((_R, 1), lambda i: (i, 0)),
            pl.BlockSpec((3, d, d_out), lambda i: (0, 0, 0)),
            pl.BlockSpec((1, d_out), lambda i: (0, 0)),
            pl.BlockSpec((1, d_out), lambda i: (0, 0)),
        ],
        out_specs=[
            pl.BlockSpec((_R, d_out), lambda i: (i, 0)),
            pl.BlockSpec((_R, d_out), lambda i: (i, 0)),
        ],
        out_shape=[
            jax.ShapeDtypeStruct((_N, d_out), jnp.float32),
            jax.ShapeDtypeStruct((_N, d_out), jnp.float32),
        ],
    )(h, t1, p2, dis, wstk, scale, shift)


def _tc_final(h, t1, p2, dis, wstk, scale, shift, batch2d, wl, bl, d, d_out):
    def body(h_ref, t1_ref, p2_ref, dis_ref, w_ref, sc_ref, sh_ref,
             bt_ref, wl_ref, bl_ref, out_ref, sums, counts):
        i = pl.program_id(0)

        @pl.when(i == 0)
        def _():
            sums[...] = jnp.zeros_like(sums)
            counts[...] = jnp.zeros_like(counts)

        dis_v = dis_ref[...]
        h3 = _cheb_block(h_ref[...], t1_ref[...], p2_ref, dis_v, w_ref,
                         sc_ref, sh_ref, False, d)
        gids = lax.broadcasted_iota(jnp.int32, (1, _NG), 1)
        mask = (bt_ref[...] == gids).astype(jnp.float32)
        sums[...] += lax.dot_general(
            mask, h3, (((0,), (0,)), ((), ())),
            preferred_element_type=jnp.float32,
            precision=lax.Precision.HIGHEST)
        ones_col = jnp.ones((_R, 1), jnp.float32)
        counts[...] += lax.dot_general(
            mask, ones_col, (((0,), (0,)), ((), ())),
            preferred_element_type=jnp.float32,
            precision=lax.Precision.HIGHEST)

        @pl.when(i == _GRID - 1)
        def _():
            mean = sums[...] / jnp.maximum(counts[...], 1.0)
            out_ref[...] = jnp.dot(
                mean, wl_ref[...], preferred_element_type=jnp.float32
            ) + bl_ref[...]

    return pl.pallas_call(
        body,
        grid=(_GRID,),
        in_specs=[
            pl.BlockSpec((_R, d), lambda i: (i, 0)),
            pl.BlockSpec((_R, d), lambda i: (i, 0)),
            pl.BlockSpec((2, _R, d), lambda i: (0, i, 0)),
            pl.BlockSpec((_R, 1), lambda i: (i, 0)),
            pl.BlockSpec((3, d, d_out), lambda i: (0, 0, 0)),
            pl.BlockSpec((1, d_out), lambda i: (0, 0)),
            pl.BlockSpec((1, d_out), lambda i: (0, 0)),
            pl.BlockSpec((_R, 1), lambda i: (i, 0)),
            pl.BlockSpec((d_out, 2), lambda i: (0, 0)),
            pl.BlockSpec((1, 2), lambda i: (0, 0)),
        ],
        out_specs=pl.BlockSpec((_NG, 2), lambda i: (0, 0)),
        out_shape=jax.ShapeDtypeStruct((_NG, 2), jnp.float32),
        scratch_shapes=[
            pltpu.VMEM((_NG, d_out), jnp.float32),
            pltpu.VMEM((_NG, 1), jnp.float32),
        ],
    )(h, t1, p2, dis, wstk, scale, shift, batch2d, wl, bl)


def _bn_fold(b, g, be, rm, rv):
    s = g / jnp.sqrt(rv + _EPS)
    return s[None, :], (b * s + be - rm * s)[None, :]


@jax.jit
def kernel(x, edge_index, batch,
           W1, b1, g1, be1, rm1, rv1,
           W2, b2, g2, be2, rm2, rv2,
           W3, b3, g3, be3, rm3, rv3,
           Wl, bl):
    row2d = edge_index[0].reshape(_NCH_TOT, _C)
    col2d = edge_index[1].reshape(_NCH_TOT, _C)

    sc1, sh1 = _bn_fold(b1, g1, be1, rm1, rv1)
    sc2, sh2 = _bn_fold(b2, g2, be2, rm2, rv2)
    sc3, sh3 = _bn_fold(b3, g3, be3, rm3, rv3)

    degp = _make_sc_deg()(row2d)
    u1, dis = _tc_start(x, degp, 64)

    p = _make_sc_pass(64, True)(u1, row2d, col2d)
    t1, u2 = _tc_mid_halves(p, dis, 64)
    p2 = _make_sc_pass(64, True)(u2, row2d, col2d)
    h, un = _tc_end(x, t1, p2, dis, W1, sc1, sh1, True, 128, 64)

    p = _make_sc_pass(64, False)(un, row2d, col2d)
    t1, u2 = _tc_mid(p, dis, 64)
    p2 = _make_sc_pass(64, False)(u2, row2d, col2d)
    h, un = _tc_end(h, t1, p2, dis, W2, sc2, sh2, False, 64, 32)

    p = _make_sc_pass(32, False)(un, row2d, col2d)
    t1, u2 = _tc_mid(p, dis, 32)
    p2 = _make_sc_pass(32, False)(u2, row2d, col2d)

    batch2d = batch.reshape(_N, 1)
    return _tc_final(h, t1, p2, dis, W3, sc3, sh3, batch2d,
                     Wl, bl[None, :], 32, 16)


# speedup vs baseline: 1.0275x; 1.0275x over previous
import functools

import jax
import jax.numpy as jnp
from jax import lax
from jax.experimental import pallas as pl
from jax.experimental.pallas import tpu as pltpu
from jax.experimental.pallas import tpu_sc as plsc

_N = 10000
_NPAD = 10240
_E = 320000
_NG = 16
_EPS = 1e-5

_C = 125
_NCH_TOT = _E // _C
_NW = 32
_NCHW = _NCH_TOT // _NW
_RPS = _NPAD // 16
_ZC = 128
_DEG_D = 16

_R = 2000
_GRID = _N // _R


def _fill_f32(buf, rows, d, val):
    vals = jnp.full((16,), val, jnp.float32)

    def row_body(i, _):
        def col_body(k, _):
            buf[i, pl.ds(k * 16, 16)] = vals
            return 0
        return lax.fori_loop(0, d // 16, col_body, 0)

    lax.fori_loop(0, rows, row_body, 0)


def _edge_loop(table, acc, rowi, coli, bufs, gsems, ssems, nbuf, nch):
    b0s = [buf.at[pl.ds(0, _C)] for buf in bufs]
    for b in range(nbuf):
        pltpu.async_copy(table.at[rowi.at[b]], b0s[b], gsems[b])

    def grp(it, _):
        base = it * nbuf
        for b in range(nbuf):
            pltpu.make_async_copy(
                table.at[rowi.at[0]], b0s[b], gsems[b]).wait()
            pltpu.async_copy(b0s[b], acc.at[coli.at[base + b]],
                             ssems[b], add=True)
        nxt = base + nbuf
        for b in range(nbuf):
            pltpu.make_async_copy(
                b0s[b], acc.at[coli.at[0]], ssems[b]).wait()

            @pl.when(nxt + b < nch)
            def _():
                pltpu.async_copy(table.at[rowi.at[nxt + b]],
                                 b0s[b], gsems[b])
        return 0

    lax.fori_loop(0, nch // nbuf, grp, 0)


def _zero_acc(acc, buf0, s, D):
    _fill_f32(buf0, _ZC, D, 0.0)
    for k in range(_RPS // _ZC):
        pltpu.sync_copy(buf0, acc.at[pl.ds(s * _RPS + k * _ZC, _ZC)])


def _sc_edge_body(D, nbuf, deg_mode, fused, u_hbm, col_hbm, row_hbm,
                  out_hbm, refs):
    nch = 2 * _NCHW if fused else _NCHW
    rowi, coli = refs[0], refs[1]
    bufs = list(refs[2:2 + nbuf])
    acc = refs[2 + nbuf]
    gsems = list(refs[3 + nbuf:3 + 2 * nbuf])
    ssems = list(refs[3 + 2 * nbuf:3 + 3 * nbuf])

    c = lax.axis_index("c")
    s = lax.axis_index("s")
    idx_base = s * nch if fused else (s * 2 + c) * nch
    table = u_hbm.at[c] if fused else u_hbm

    ld0 = pltpu.async_copy(col_hbm.at[pl.ds(idx_base, nch)], coli, ssems[0])
    if not deg_mode:
        ld1 = pltpu.async_copy(row_hbm.at[pl.ds(idx_base, nch)], rowi,
                               ssems[1])

    _zero_acc(acc, bufs[0], s, D)
    ld0.wait()
    if not deg_mode:
        ld1.wait()
    plsc.subcore_barrier()

    if deg_mode:
        _fill_f32(bufs[0], _C, D, 1.0)
        b00 = bufs[0].at[pl.ds(0, _C)]

        def grp(it, _):
            base = it * nbuf
            for b in range(nbuf):
                pltpu.async_copy(b00, acc.at[coli.at[base + b]],
                                 ssems[b], add=True)
            for b in range(nbuf):
                pltpu.make_async_copy(
                    b00, acc.at[coli.at[0]], ssems[b]).wait()
            return 0

        lax.fori_loop(0, nch // nbuf, grp, 0)
    else:
        _edge_loop(table, acc, rowi, coli, bufs, gsems, ssems, nbuf, nch)
    plsc.subcore_barrier()

    sl = pl.ds(s * _RPS, _RPS)
    pltpu.sync_copy(acc.at[sl], out_hbm.at[c].at[sl])


def _sc_scratch(D, nbuf, nch):
    return ([
        pltpu.VMEM((nch, _C), jnp.int32),
        pltpu.VMEM((nch, _C), jnp.int32),
    ] + [pltpu.VMEM((_ZC, D), jnp.float32)] * nbuf
      + [pltpu.VMEM_SHARED((_NPAD, D), jnp.float32)]
      + [pltpu.SemaphoreType.DMA] * (2 * nbuf))


_FUSED_NBUF = {64: 5, 32: 8, 16: 8}
_Q_NBUF = 8
_LAYER_NBUF = {64: 4, 32: 8, 16: 8}


@functools.cache
def _make_sc_pass(D, fused):
    mesh = plsc.VectorSubcoreMesh(core_axis_name="c", subcore_axis_name="s")
    nbuf = _FUSED_NBUF[D] if fused else _Q_NBUF
    nch = 2 * _NCHW if fused else _NCHW

    @functools.partial(
        pl.kernel, mesh=mesh,
        out_type=jax.ShapeDtypeStruct((2, _NPAD, D), jnp.float32),
        scratch_types=_sc_scratch(D, nbuf, nch),
        compiler_params=pltpu.CompilerParams(use_tc_tiling_on_sc=False),
        name=f"sc_edge_pass_{D}_{'ab' if fused else 'q'}",
    )
    def sc_pass(u_hbm, row_hbm, col_hbm, out_hbm, *refs):
        _sc_edge_body(D, nbuf, False, fused, u_hbm, col_hbm, row_hbm,
                      out_hbm, refs)

    return sc_pass


@functools.cache
def _make_sc_layer(D):
    mesh = plsc.VectorSubcoreMesh(core_axis_name="c", subcore_axis_name="s")
    nbuf = _LAYER_NBUF[D]
    nch = 2 * _NCHW

    @functools.partial(
        pl.kernel, mesh=mesh,
        out_type=[
            jax.ShapeDtypeStruct((2, _NPAD, D), jnp.float32),
            jax.ShapeDtypeStruct((2, _NPAD, D), jnp.float32),
            jax.ShapeDtypeStruct((2, _NPAD, D), jnp.float32),
        ],
        scratch_types=_sc_scratch(D, nbuf, nch)
        + [pltpu.VMEM((_RPS, _DEG_D), jnp.float32)],
        compiler_params=pltpu.CompilerParams(use_tc_tiling_on_sc=False),
        name=f"sc_layer_{D}",
    )
    def sc_layer(u1_hbm, nd16_hbm, row_hbm, col_hbm,
                 v1_hbm, u2_hbm, v2_hbm, *refs):
        rowi, coli = refs[0], refs[1]
        bufs = list(refs[2:2 + nbuf])
        acc = refs[2 + nbuf]
        gsems = list(refs[3 + nbuf:3 + 2 * nbuf])
        ssems = list(refs[3 + 2 * nbuf:3 + 3 * nbuf])
        nd16 = refs[3 + 3 * nbuf]

        c = lax.axis_index("c")
        s = lax.axis_index("s")
        sl = pl.ds(s * _RPS, _RPS)

        ld0 = pltpu.async_copy(col_hbm.at[pl.ds(s * nch, nch)], coli,
                               ssems[0])
        ld1 = pltpu.async_copy(row_hbm.at[pl.ds(s * nch, nch)], rowi,
                               ssems[1])
        ld2 = pltpu.async_copy(nd16_hbm.at[sl], nd16, gsems[0])
        _zero_acc(acc, bufs[0], s, D)
        ld0.wait()
        ld1.wait()
        ld2.wait()
        plsc.subcore_barrier()

        _edge_loop(u1_hbm.at[c], acc, rowi, coli, bufs, gsems, ssems,
                   nbuf, nch)
        plsc.subcore_barrier()

        pltpu.sync_copy(acc.at[sl], v1_hbm.at[c].at[sl])
        buf = bufs[0]
        for k in range(_RPS // _ZC):
            ksl = pl.ds(s * _RPS + k * _ZC, _ZC)
            pltpu.sync_copy(acc.at[ksl], buf)

            def srow(r, _):
                nd = nd16[k * _ZC + r, pl.ds(0, 16)]
                for t in range(D // 16):
                    buf[r, pl.ds(t * 16, 16)] = buf[r, pl.ds(t * 16, 16)] * nd
                return 0

            lax.fori_loop(0, _ZC, srow, 0)
            pltpu.sync_copy(buf, u2_hbm.at[c].at[ksl])

        _zero_acc(acc, bufs[0], s, D)
        plsc.subcore_barrier()
        _edge_loop(u2_hbm.at[c], acc, rowi, coli, bufs, gsems, ssems,
                   nbuf, nch)
        plsc.subcore_barrier()

        pltpu.sync_copy(acc.at[sl], v2_hbm.at[c].at[sl])

    return sc_layer


@functools.cache
def _make_sc_deg():
    mesh = plsc.VectorSubcoreMesh(core_axis_name="c", subcore_axis_name="s")
    D = _DEG_D

    @functools.partial(
        pl.kernel, mesh=mesh,
        out_type=jax.ShapeDtypeStruct((2, _NPAD, D), jnp.float32),
        scratch_types=_sc_scratch(D, _Q_NBUF, _NCHW),
        compiler_params=pltpu.CompilerParams(use_tc_tiling_on_sc=False),
        name="sc_deg_pass",
    )
    def sc_deg(row_hbm, out_hbm, *refs):
        _sc_edge_body(D, _Q_NBUF, True, False, None, row_hbm, None,
                      out_hbm, refs)

    return sc_deg


def _lrelu(z):
    return jnp.where(z >= 0, z, 0.01 * z)


def _tc_start(x, degp, d_half):
    def body(x_ref, d_ref, u_ref, dis_ref, nd_ref):
        deg = d_ref[0, :, 0:1] + d_ref[1, :, 0:1]
        dis = jnp.where(deg > 0, lax.rsqrt(deg), 0.0)
        u_ref[0] = dis * x_ref[:, :d_half]
        u_ref[1] = dis * x_ref[:, d_half:]
        dis_ref[...] = dis
        nd_ref[...] = jnp.broadcast_to(-(dis * dis), (_R, _DEG_D))

    f_in = x.shape[1]
    return pl.pallas_call(
        body,
        grid=(_GRID,),
        in_specs=[
            pl.BlockSpec((_R, f_in), lambda i: (i, 0)),
            pl.BlockSpec((2, _R, _DEG_D), lambda i: (0, i, 0)),
        ],
        out_specs=[
            pl.BlockSpec((2, _R, d_half), lambda i: (0, i, 0)),
            pl.BlockSpec((_R, 1), lambda i: (i, 0)),
            pl.BlockSpec((_R, _DEG_D), lambda i: (i, 0)),
        ],
        out_shape=[
            jax.ShapeDtypeStruct((2, _N, d_half), jnp.float32),
            jax.ShapeDtypeStruct((_N, 1), jnp.float32),
            jax.ShapeDtypeStruct((_NPAD, _DEG_D), jnp.float32),
        ],
    )(x, degp)


def _cheb_block(h, p1_ref, p2_ref, dis_v, w_ref, sc_ref, sh_ref):
    t1 = -dis_v * jnp.concatenate([p1_ref[0], p1_ref[1]], axis=1)
    v2 = jnp.concatenate([p2_ref[0], p2_ref[1]], axis=1)
    tx2 = 2.0 * (-dis_v * v2) - h
    mm = (jnp.dot(h, w_ref[0], preferred_element_type=jnp.float32)
          + jnp.dot(t1, w_ref[1], preferred_element_type=jnp.float32)
          + jnp.dot(tx2, w_ref[2], preferred_element_type=jnp.float32))
    return _lrelu(mm * sc_ref[...] + sh_ref[...])


def _tc_end(h, p1, p2, dis, wstk, scale, shift, d, d_out):
    d_p = d // 2

    def body(h_ref, p1_ref, p2_ref, dis_ref, w_ref, sc_ref, sh_ref,
             hn_ref, un_ref):
        dis_v = dis_ref[...]
        hn = _cheb_block(h_ref[...], p1_ref, p2_ref, dis_v, w_ref,
                         sc_ref, sh_ref)
        hn_ref[...] = hn
        un_ref[0] = dis_v * hn[:, :d_out // 2]
        un_ref[1] = dis_v * hn[:, d_out // 2:]

    return pl.pallas_call(
        body,
        grid=(_GRID,),
        in_specs=[
            pl.BlockSpec((_R, d), lambda i: (i, 0)),
            pl.BlockSpec((2, _R, d_p), lambda i: (0, i, 0)),
            pl.BlockSpec((2, _R, d_p), lambda i: (0, i, 0)),
            pl.BlockSpec((_R, 1), lambda i: (i, 0)),
            pl.BlockSpec((3, d, d_out), lambda i: (0, 0, 0)),
            pl.BlockSpec((1, d_out), lambda i: (0, 0)),
            pl.BlockSpec((1, d_out), lambda i: (0, 0)),
        ],
        out_specs=[
            pl.BlockSpec((_R, d_out), lambda i: (i, 0)),
            pl.BlockSpec((2, _R, d_out // 2), lambda i: (0, i, 0)),
        ],
        out_shape=[
            jax.ShapeDtypeStruct((_N, d_out), jnp.float32),
            jax.ShapeDtypeStruct((2, _N, d_out // 2), jnp.float32),
        ],
    )(h, p1, p2, dis, wstk, scale, shift)


def _tc_final(h, p1, p2, dis, wstk, scale, shift, batch2d, wl, bl,
              d, d_out):
    d_p = d // 2

    def body(h_ref, p1_ref, p2_ref, dis_ref, w_ref, sc_ref, sh_ref,
             bt_ref, wl_ref, bl_ref, out_ref, sums, counts):
        i = pl.program_id(0)

        @pl.when(i == 0)
        def _():
            sums[...] = jnp.zeros_like(sums)
            counts[...] = jnp.zeros_like(counts)

        dis_v = dis_ref[...]
        h3 = _cheb_block(h_ref[...], p1_ref, p2_ref, dis_v, w_ref,
                         sc_ref, sh_ref)
        gids = lax.broadcasted_iota(jnp.int32, (1, _NG), 1)
        mask = (bt_ref[...] == gids).astype(jnp.float32)
        sums[...] += lax.dot_general(
            mask, h3, (((0,), (0,)), ((), ())),
            preferred_element_type=jnp.float32,
            precision=lax.Precision.HIGHEST)
        ones_col = jnp.ones((_R, 1), jnp.float32)
        counts[...] += lax.dot_general(
            mask, ones_col, (((0,), (0,)), ((), ())),
            preferred_element_type=jnp.float32,
            precision=lax.Precision.HIGHEST)

        @pl.when(i == _GRID - 1)
        def _():
            mean = sums[...] / jnp.maximum(counts[...], 1.0)
            out_ref[...] = jnp.dot(
                mean, wl_ref[...], preferred_element_type=jnp.float32
            ) + bl_ref[...]

    return pl.pallas_call(
        body,
        grid=(_GRID,),
        in_specs=[
            pl.BlockSpec((_R, d), lambda i: (i, 0)),
            pl.BlockSpec((2, _R, d_p), lambda i: (0, i, 0)),
            pl.BlockSpec((2, _R, d_p), lambda i: (0, i, 0)),
            pl.BlockSpec((_R, 1), lambda i: (i, 0)),
            pl.BlockSpec((3, d, d_out), lambda i: (0, 0, 0)),
            pl.BlockSpec((1, d_out), lambda i: (0, 0)),
            pl.BlockSpec((1, d_out), lambda i: (0, 0)),
            pl.BlockSpec((_R, 1), lambda i: (i, 0)),
            pl.BlockSpec((d_out, 2), lambda i: (0, 0)),
            pl.BlockSpec((1, 2), lambda i: (0, 0)),
        ],
        out_specs=pl.BlockSpec((_NG, 2), lambda i: (0, 0)),
        out_shape=jax.ShapeDtypeStruct((_NG, 2), jnp.float32),
        scratch_shapes=[
            pltpu.VMEM((_NG, d_out), jnp.float32),
            pltpu.VMEM((_NG, 1), jnp.float32),
        ],
    )(h, p1, p2, dis, wstk, scale, shift, batch2d, wl, bl)


def _bn_fold(b, g, be, rm, rv):
    s = g / jnp.sqrt(rv + _EPS)
    return s[None, :], (b * s + be - rm * s)[None, :]


@jax.jit
def kernel(x, edge_index, batch,
           W1, b1, g1, be1, rm1, rv1,
           W2, b2, g2, be2, rm2, rv2,
           W3, b3, g3, be3, rm3, rv3,
           Wl, bl):
    row2d = edge_index[0].reshape(_NCH_TOT, _C)
    col2d = edge_index[1].reshape(_NCH_TOT, _C)

    sc1, sh1 = _bn_fold(b1, g1, be1, rm1, rv1)
    sc2, sh2 = _bn_fold(b2, g2, be2, rm2, rv2)
    sc3, sh3 = _bn_fold(b3, g3, be3, rm3, rv3)

    degp = _make_sc_deg()(row2d)
    u1, dis, nd16 = _tc_start(x, degp, 64)

    p1, _, p2 = _make_sc_layer(64)(u1, nd16, row2d, col2d)
    h, un = _tc_end(x, p1, p2, dis, W1, sc1, sh1, 128, 64)

    p1, _, p2 = _make_sc_layer(32)(un, nd16, row2d, col2d)
    h, un = _tc_end(h, p1, p2, dis, W2, sc2, sh2, 64, 32)

    p1, _, p2 = _make_sc_layer(16)(un, nd16, row2d, col2d)

    batch2d = batch.reshape(_N, 1)
    return _tc_final(h, p1, p2, dis, W3, sc3, sh3, batch2d,
                     Wl, bl[None, :], 32, 16)
